# Initial kernel scaffold; baseline (speedup 1.0000x reference)
import functools

import jax
import jax.numpy as jnp
from jax import lax
from jax.experimental import pallas as pl
from jax.experimental.pallas import tpu as pltpu
from jax.experimental.pallas import tpu_sc as plsc

N = 10000
E = 320000
NPAD = 10240
D = 128
NC, NS = 2, 16
NW = NC * NS
EPT = E // NW
CH = 80
SLICE = NPAD // NS
RB = 1024
GRID = NPAD // RB

_mesh = plsc.VectorSubcoreMesh(core_axis_name="c", subcore_axis_name="s")


@functools.partial(
    pl.kernel,
    out_type=jax.ShapeDtypeStruct((NC, NPAD), jnp.float32),
    mesh=_mesh,
    scratch_types=[
        pltpu.VMEM((NPAD,), jnp.float32),
        pltpu.VMEM((CH,), jnp.int32),
        pltpu.VMEM((SLICE,), jnp.float32),
        pltpu.VMEM((SLICE,), jnp.float32),
        pltpu.VMEM_SHARED((NS, NPAD), jnp.float32),
    ],
)
def _deg_sc(dst_hbm, deg_hbm, hist_v, dstb_v, acc_v, tmp_v, hists_sh):
    c = lax.axis_index("c")
    s = lax.axis_index("s")
    wid = c * NS + s
    zero16 = jnp.zeros((16,), jnp.float32)
    ones16 = jnp.ones((16,), jnp.float32)

    @pl.loop(0, NPAD, step=16)
    def _(i):
        hist_v[pl.ds(i, 16)] = zero16

    base = wid * EPT

    @pl.loop(0, EPT, step=CH)
    def _(j):
        pltpu.sync_copy(dst_hbm.at[pl.ds(base + j, CH)], dstb_v)

        @pl.loop(0, CH, step=16)
        def _(k):
            plsc.addupdate_scatter(hist_v, [dstb_v[pl.ds(k, 16)]], ones16)

    pltpu.sync_copy(hist_v, hists_sh.at[s])
    plsc.subcore_barrier()

    sl = s * SLICE

    @pl.loop(0, SLICE, step=16)
    def _(k):
        acc_v[pl.ds(k, 16)] = zero16

    @pl.loop(0, NS)
    def _(r):
        pltpu.sync_copy(hists_sh.at[r, pl.ds(sl, SLICE)], tmp_v)

        @pl.loop(0, SLICE, step=16)
        def _(k):
            acc_v[pl.ds(k, 16)] = acc_v[pl.ds(k, 16)] + tmp_v[pl.ds(k, 16)]

    pltpu.sync_copy(acc_v, deg_hbm.at[c, pl.ds(sl, SLICE)])


@functools.partial(
    pl.kernel,
    out_type=jax.ShapeDtypeStruct((NC, NPAD, D), jnp.float32),
    mesh=_mesh,
    scratch_types=[
        pltpu.VMEM((CH,), jnp.int32),
        pltpu.VMEM((CH,), jnp.int32),
        pltpu.VMEM((CH, D), jnp.float32),
        pltpu.VMEM((64, D), jnp.float32),
        pltpu.VMEM_SHARED((NPAD, D), jnp.float32),
    ],
)
def _prop_sc(src_hbm, dst_hbm, u_hbm, out_hbm, srci_v, dsti_v, rows_v,
             bounce_v, acc_sh):
    c = lax.axis_index("c")
    s = lax.axis_index("s")
    wid = c * NS + s
    zero16 = jnp.zeros((16,), jnp.float32)

    @pl.loop(0, 64)
    def _(r):
        @pl.loop(0, D, step=16)
        def _(k):
            bounce_v[r, pl.ds(k, 16)] = zero16

    sl = s * SLICE

    @pl.loop(0, SLICE, step=64)
    def _(r):
        pltpu.sync_copy(bounce_v, acc_sh.at[pl.ds(sl + r, 64)])

    plsc.subcore_barrier()

    base = wid * EPT

    @pl.loop(0, EPT, step=CH)
    def _(j):
        pltpu.sync_copy(src_hbm.at[pl.ds(base + j, CH)], srci_v)
        pltpu.sync_copy(dst_hbm.at[pl.ds(base + j, CH)], dsti_v)
        pltpu.sync_copy(u_hbm.at[srci_v], rows_v)
        pltpu.sync_copy(rows_v, acc_sh.at[dsti_v], add=True)

    plsc.subcore_barrier()

    @pl.loop(0, SLICE, step=64)
    def _(r):
        pltpu.sync_copy(acc_sh.at[pl.ds(sl + r, 64)], bounce_v)
        pltpu.sync_copy(bounce_v, out_hbm.at[c, pl.ds(sl + r, 64)])


def _dinv_of(degp):
    deg = degp[0] + degp[1] + 1.0
    return lax.rsqrt(deg), deg


def _pre_body(degp_ref, x_ref, o_ref):
    dinv, _ = _dinv_of(degp_ref[...])
    o_ref[...] = dinv * x_ref[...]


def _comb_body(degp_ref, p_ref, u_ref, o_ref):
    _, deg = _dinv_of(degp_ref[...])
    s = p_ref[0] + p_ref[1] + u_ref[...]
    o_ref[...] = s / deg


def _mlp_body(degp_ref, p_ref, u_ref, w1_ref, b1_ref, w2_ref, o_ref):
    dinv, _ = _dinv_of(degp_ref[...])
    hin = dinv * (p_ref[0] + p_ref[1] + u_ref[...])
    h = jnp.dot(hin, w1_ref[...], preferred_element_type=jnp.float32)
    h = jnp.maximum(h + b1_ref[...], 0.0)
    y = jnp.dot(h, w2_ref[...], preferred_element_type=jnp.float32)
    o_ref[...] = dinv * y


def _final_body(degp_ref, p_ref, u_ref, b2_ref, o_ref):
    dinv, _ = _dinv_of(degp_ref[...])
    z = dinv * (p_ref[0] + p_ref[1] + u_ref[...]) + b2_ref[...]
    m = jnp.max(z, axis=1, keepdims=True)
    e = jnp.exp(z - m)
    lse = jnp.log(jnp.sum(e, axis=1, keepdims=True))
    o_ref[...] = z - m - lse


_degp_spec = pl.BlockSpec((NC, RB, 1), lambda i: (0, i, 0))
_row_spec = pl.BlockSpec((RB, D), lambda i: (i, 0))
_p_spec = pl.BlockSpec((NC, RB, D), lambda i: (0, i, 0))


def _rows_out(d=D):
    return jax.ShapeDtypeStruct((NPAD, d), jnp.float32)


_tc_pre = pl.pallas_call(
    _pre_body, grid=(GRID,),
    in_specs=[_degp_spec, _row_spec],
    out_specs=_row_spec, out_shape=_rows_out())

_tc_comb = pl.pallas_call(
    _comb_body, grid=(GRID,),
    in_specs=[_degp_spec, _p_spec, _row_spec],
    out_specs=_row_spec, out_shape=_rows_out())

_tc_mlp = pl.pallas_call(
    _mlp_body, grid=(GRID,),
    in_specs=[_degp_spec, _p_spec, _row_spec,
              pl.BlockSpec((D, 256), lambda i: (0, 0)),
              pl.BlockSpec((1, 256), lambda i: (0, 0)),
              pl.BlockSpec((256, D), lambda i: (0, 0))],
    out_specs=_row_spec, out_shape=_rows_out())

_tc_final = pl.pallas_call(
    _final_body, grid=(GRID,),
    in_specs=[_degp_spec, _p_spec, _row_spec,
              pl.BlockSpec((1, D), lambda i: (0, 0))],
    out_specs=_row_spec, out_shape=_rows_out())


def kernel(x, edge_index, W1, b1, W2, b2):
    src = edge_index[0]
    dst = edge_index[1]
    x_pad = jnp.pad(x, ((0, NPAD - N), (0, 0)))

    degp = _deg_sc(dst).reshape(NC, NPAD, 1)

    u0 = _tc_pre(degp, x_pad)
    p = _prop_sc(src, dst, u0)
    u1 = _tc_comb(degp, p, u0)
    p = _prop_sc(src, dst, u1)
    u2 = _tc_mlp(degp, p, u1, W1.T, b1.reshape(1, -1), W2.T)
    p = _prop_sc(src, dst, u2)
    u3 = _tc_comb(degp, p, u2)
    p = _prop_sc(src, dst, u3)
    out = _tc_final(degp, p, u3, b2.reshape(1, -1))
    return out[:N]


# speedup vs baseline: 11.4239x; 11.4239x over previous
import dataclasses
import functools

import jax
import jax.numpy as jnp
from jax import lax
from jax.experimental import pallas as pl
from jax.experimental.pallas import tpu as pltpu
from jax.experimental.pallas import tpu_sc as plsc

N = 10000
E = 320000
NPAD = 10240
D = 128
NC, NS = 2, 16
NW = NC * NS
EPT = E // NW
CH = 80
SLICE = NPAD // NS
RB = 1024
GRID = NPAD // RB

_mesh = plsc.VectorSubcoreMesh(core_axis_name="c", subcore_axis_name="s")

_sc_params = pltpu.CompilerParams()
if "needs_layout_passes" in pltpu.CompilerParams.__dataclass_fields__:
    _sc_params = dataclasses.replace(_sc_params, needs_layout_passes=False)


@functools.partial(
    pl.kernel,
    out_type=jax.ShapeDtypeStruct((NC, NPAD), jnp.float32),
    mesh=_mesh,
    compiler_params=_sc_params,
    scratch_types=[
        pltpu.VMEM((NPAD,), jnp.float32),
        pltpu.VMEM((CH,), jnp.int32),
        pltpu.VMEM((SLICE,), jnp.float32),
        pltpu.VMEM((SLICE,), jnp.float32),
        pltpu.VMEM_SHARED((NS, NPAD), jnp.float32),
    ],
)
def _deg_sc(dst_hbm, deg_hbm, hist_v, dstb_v, acc_v, tmp_v, hists_sh):
    c = lax.axis_index("c")
    s = lax.axis_index("s")
    wid = c * NS + s
    zero16 = jnp.zeros((16,), jnp.float32)
    ones16 = jnp.ones((16,), jnp.float32)

    @pl.loop(0, NPAD, step=16)
    def _(i):
        hist_v[pl.ds(i, 16)] = zero16

    base = wid * EPT

    @pl.loop(0, EPT, step=CH)
    def _(j):
        pltpu.sync_copy(dst_hbm.at[pl.ds(base + j, CH)], dstb_v)

        @pl.loop(0, CH, step=16)
        def _(k):
            plsc.addupdate_scatter(hist_v, [dstb_v[pl.ds(k, 16)]], ones16)

    pltpu.sync_copy(hist_v, hists_sh.at[s])
    plsc.subcore_barrier()

    sl = s * SLICE

    @pl.loop(0, SLICE, step=16)
    def _(k):
        acc_v[pl.ds(k, 16)] = zero16

    @pl.loop(0, NS)
    def _(r):
        pltpu.sync_copy(hists_sh.at[r, pl.ds(sl, SLICE)], tmp_v)

        @pl.loop(0, SLICE, step=16)
        def _(k):
            acc_v[pl.ds(k, 16)] = acc_v[pl.ds(k, 16)] + tmp_v[pl.ds(k, 16)]

    pltpu.sync_copy(acc_v, deg_hbm.at[c, pl.ds(sl, SLICE)])


@functools.partial(
    pl.kernel,
    out_type=jax.ShapeDtypeStruct((NC, NPAD, D), jnp.float32),
    mesh=_mesh,
    scratch_types=[
        pltpu.VMEM((CH,), jnp.int32),
        pltpu.VMEM((CH,), jnp.int32),
        pltpu.VMEM((CH, D), jnp.float32),
        pltpu.VMEM((64, D), jnp.float32),
        pltpu.VMEM_SHARED((NPAD, D), jnp.float32),
    ],
)
def _prop_sc(src_hbm, dst_hbm, u_hbm, out_hbm, srci_v, dsti_v, rows_v,
             bounce_v, acc_sh):
    c = lax.axis_index("c")
    s = lax.axis_index("s")
    wid = c * NS + s
    zero16 = jnp.zeros((16,), jnp.float32)

    @pl.loop(0, 64)
    def _(r):
        @pl.loop(0, D, step=16)
        def _(k):
            bounce_v[r, pl.ds(k, 16)] = zero16

    sl = s * SLICE

    @pl.loop(0, SLICE, step=64)
    def _(r):
        pltpu.sync_copy(bounce_v, acc_sh.at[pl.ds(sl + r, 64)])

    plsc.subcore_barrier()

    base = wid * EPT

    @pl.loop(0, EPT, step=CH)
    def _(j):
        pltpu.sync_copy(src_hbm.at[pl.ds(base + j, CH)], srci_v)
        pltpu.sync_copy(dst_hbm.at[pl.ds(base + j, CH)], dsti_v)
        pltpu.sync_copy(u_hbm.at[srci_v], rows_v)
        pltpu.sync_copy(rows_v, acc_sh.at[dsti_v], add=True)

    plsc.subcore_barrier()

    @pl.loop(0, SLICE, step=64)
    def _(r):
        pltpu.sync_copy(acc_sh.at[pl.ds(sl + r, 64)], bounce_v)
        pltpu.sync_copy(bounce_v, out_hbm.at[c, pl.ds(sl + r, 64)])


def _dinv_of(degp):
    deg = degp[0] + degp[1] + 1.0
    return lax.rsqrt(deg), deg


def _pre_body(degp_ref, x_ref, o_ref):
    dinv, _ = _dinv_of(degp_ref[...])
    o_ref[...] = dinv * x_ref[...]


def _comb_body(degp_ref, p_ref, u_ref, o_ref):
    _, deg = _dinv_of(degp_ref[...])
    s = p_ref[0] + p_ref[1] + u_ref[...]
    o_ref[...] = s / deg


def _mlp_body(degp_ref, p_ref, u_ref, w1_ref, b1_ref, w2_ref, o_ref):
    dinv, _ = _dinv_of(degp_ref[...])
    hin = dinv * (p_ref[0] + p_ref[1] + u_ref[...])
    h = jnp.dot(hin, w1_ref[...], preferred_element_type=jnp.float32)
    h = jnp.maximum(h + b1_ref[...], 0.0)
    y = jnp.dot(h, w2_ref[...], preferred_element_type=jnp.float32)
    o_ref[...] = dinv * y


def _final_body(degp_ref, p_ref, u_ref, b2_ref, o_ref):
    dinv, _ = _dinv_of(degp_ref[...])
    z = dinv * (p_ref[0] + p_ref[1] + u_ref[...]) + b2_ref[...]
    m = jnp.max(z, axis=1, keepdims=True)
    e = jnp.exp(z - m)
    lse = jnp.log(jnp.sum(e, axis=1, keepdims=True))
    o_ref[...] = z - m - lse


_degp_spec = pl.BlockSpec((NC, RB, 1), lambda i: (0, i, 0))
_row_spec = pl.BlockSpec((RB, D), lambda i: (i, 0))
_p_spec = pl.BlockSpec((NC, RB, D), lambda i: (0, i, 0))


def _rows_out(d=D):
    return jax.ShapeDtypeStruct((NPAD, d), jnp.float32)


_tc_pre = pl.pallas_call(
    _pre_body, grid=(GRID,),
    in_specs=[_degp_spec, _row_spec],
    out_specs=_row_spec, out_shape=_rows_out())

_tc_comb = pl.pallas_call(
    _comb_body, grid=(GRID,),
    in_specs=[_degp_spec, _p_spec, _row_spec],
    out_specs=_row_spec, out_shape=_rows_out())

_tc_mlp = pl.pallas_call(
    _mlp_body, grid=(GRID,),
    in_specs=[_degp_spec, _p_spec, _row_spec,
              pl.BlockSpec((D, 256), lambda i: (0, 0)),
              pl.BlockSpec((1, 256), lambda i: (0, 0)),
              pl.BlockSpec((256, D), lambda i: (0, 0))],
    out_specs=_row_spec, out_shape=_rows_out())

_tc_final = pl.pallas_call(
    _final_body, grid=(GRID,),
    in_specs=[_degp_spec, _p_spec, _row_spec,
              pl.BlockSpec((1, D), lambda i: (0, 0))],
    out_specs=_row_spec, out_shape=_rows_out())


def kernel(x, edge_index, W1, b1, W2, b2):
    src = edge_index[0]
    dst = edge_index[1]
    x_pad = jnp.pad(x, ((0, NPAD - N), (0, 0)))

    degp = _deg_sc(dst).reshape(NC, NPAD, 1)

    u0 = _tc_pre(degp, x_pad)
    p = _prop_sc(src, dst, u0)
    u1 = _tc_comb(degp, p, u0)
    p = _prop_sc(src, dst, u1)
    u2 = _tc_mlp(degp, p, u1, W1.T, b1.reshape(1, -1), W2.T)
    p = _prop_sc(src, dst, u2)
    u3 = _tc_comb(degp, p, u2)
    p = _prop_sc(src, dst, u3)
    out = _tc_final(degp, p, u3, b2.reshape(1, -1))
    return out[:N]


# speedup vs baseline: 18.2471x; 1.5973x over previous
import dataclasses
import functools

import jax
import jax.numpy as jnp
from jax import lax
from jax.experimental import pallas as pl
from jax.experimental.pallas import tpu as pltpu
from jax.experimental.pallas import tpu_sc as plsc

N = 10000
E = 320000
NPAD = 10240
D = 128
F = D // 2
NC, NS = 2, 16
NW = NC * NS
CH = 125
NBUF = 4
EPT = E // NS
NCH = EPT // CH
DCH = 80
DEPT = E // NW
SLICE = NPAD // NS
MB = 64
RB = 1024
GRID = NPAD // RB

_mesh = plsc.VectorSubcoreMesh(core_axis_name="c", subcore_axis_name="s")

_sc_params = pltpu.CompilerParams()
if "needs_layout_passes" in pltpu.CompilerParams.__dataclass_fields__:
    _sc_params = dataclasses.replace(_sc_params, needs_layout_passes=False)
_sc_flat_params = dataclasses.replace(_sc_params, use_tc_tiling_on_sc=False)


@functools.partial(
    pl.kernel,
    out_type=jax.ShapeDtypeStruct((NW * NPAD,), jnp.float32),
    mesh=_mesh,
    compiler_params=_sc_params,
    scratch_types=[
        pltpu.VMEM((NPAD,), jnp.float32),
        pltpu.VMEM((DCH,), jnp.int32),
    ],
)
def _deg_sc(dst_hbm, deg_hbm, hist_v, dstb_v):
    c = lax.axis_index("c")
    s = lax.axis_index("s")
    wid = c * NS + s
    zero16 = jnp.zeros((16,), jnp.float32)
    ones16 = jnp.ones((16,), jnp.float32)

    @pl.loop(0, NPAD, step=16)
    def _(i):
        hist_v[pl.ds(i, 16)] = zero16

    base = wid * DEPT

    @pl.loop(0, DEPT, step=DCH)
    def _(j):
        pltpu.sync_copy(dst_hbm.at[pl.ds(base + j, DCH)], dstb_v)

        @pl.loop(0, DCH, step=16)
        def _(k):
            plsc.addupdate_scatter(hist_v, [dstb_v[pl.ds(k, 16)]], ones16)

    pltpu.sync_copy(hist_v, deg_hbm.at[pl.ds(wid * NPAD, NPAD)])


@functools.partial(
    pl.kernel,
    out_type=[jax.ShapeDtypeStruct((NC, NPAD, F), jnp.float32),
              jax.ShapeDtypeStruct((NC, NPAD, F), jnp.float32)],
    mesh=_mesh,
    compiler_params=_sc_flat_params,
    scratch_types=[
        pltpu.VMEM((NCH, CH), jnp.int32),
        pltpu.VMEM((NCH, CH), jnp.int32),
        pltpu.VMEM((CH, F), jnp.float32),
        pltpu.VMEM((CH, F), jnp.float32),
        pltpu.VMEM((CH, F), jnp.float32),
        pltpu.VMEM((CH, F), jnp.float32),
        pltpu.VMEM((MB, F), jnp.float32),
        pltpu.VMEM((MB, F), jnp.float32),
        pltpu.VMEM((MB, F), jnp.float32),
        pltpu.VMEM((MB, F), jnp.float32),
        pltpu.VMEM((MB,), jnp.float32),
        pltpu.VMEM_SHARED((NPAD, F), jnp.float32),
    ] + [pltpu.SemaphoreType.DMA] * (2 * NBUF),
)
def _prop2_sc(src_hbm, dst_hbm, ua_hbm, ub_hbm, rdeg_hbm,
              acc_hbm, mid_hbm,
              src_v, dst_v, r0, r1, r2, r3, zbuf_v, accb_v, ub_v, midb_v,
              rdb_v, acc_sh, g0, g1, g2, g3, s0, s1, s2, s3):
    c = lax.axis_index("c")
    s = lax.axis_index("s")
    rows = (r0, r1, r2, r3)
    gsem = (g0, g1, g2, g3)
    ssem = (s0, s1, s2, s3)
    zero16 = jnp.zeros((16,), jnp.float32)

    @pl.loop(0, MB)
    def _(r):
        @pl.loop(0, F, step=16)
        def _(k):
            zbuf_v[r, pl.ds(k, 16)] = zero16

    sl = s * SLICE

    @pl.loop(0, SLICE, step=MB)
    def _(r):
        pltpu.sync_copy(zbuf_v, acc_sh.at[pl.ds(sl + r, MB)])

    pltpu.sync_copy(src_hbm.at[pl.ds(s * NCH, NCH)], src_v)
    pltpu.sync_copy(dst_hbm.at[pl.ds(s * NCH, NCH)], dst_v)

    plsc.subcore_barrier()

    def edge_pass(table):
        for b in range(NBUF):
            pltpu.async_copy(table.at[src_v.at[b]], rows[b], gsem[b])

        @pl.loop(0, NCH, step=NBUF)
        def _(j):
            for b in range(NBUF):
                jj = j + b
                pltpu.make_async_copy(table.at[src_v.at[jj]], rows[b],
                                      gsem[b]).wait()
                pltpu.async_copy(rows[b], acc_sh.at[dst_v.at[jj]], ssem[b],
                                 add=True)

            for b in range(NBUF):
                jj = j + b
                pltpu.make_async_copy(rows[b], acc_sh.at[dst_v.at[jj]],
                                      ssem[b]).wait()

                @pl.when(jj + NBUF < NCH)
                def _():
                    pltpu.async_copy(table.at[src_v.at[jj + NBUF]], rows[b],
                                     gsem[b])

    @pl.when(c == 0)
    def _():
        edge_pass(ua_hbm)

    @pl.when(c == 1)
    def _():
        edge_pass(ub_hbm)

    plsc.subcore_barrier()

    @pl.loop(0, SLICE, step=MB)
    def _(r):
        row0 = sl + r
        pltpu.sync_copy(acc_sh.at[pl.ds(row0, MB)], accb_v)

        @pl.when(c == 0)
        def _():
            pltpu.sync_copy(ua_hbm.at[pl.ds(row0, MB)], ub_v)

        @pl.when(c == 1)
        def _():
            pltpu.sync_copy(ub_hbm.at[pl.ds(row0, MB)], ub_v)

        pltpu.sync_copy(rdeg_hbm.at[pl.ds(row0, MB)], rdb_v)

        @pl.loop(0, MB)
        def _(i):
            rd16 = plsc.load_gather(rdb_v, [jnp.full((16,), i, jnp.int32)])
            for k in range(0, F, 16):
                midb_v[i, pl.ds(k, 16)] = (
                    accb_v[i, pl.ds(k, 16)] + ub_v[i, pl.ds(k, 16)]) * rd16

        pltpu.sync_copy(midb_v, mid_hbm.at[c, pl.ds(row0, MB)])
        pltpu.sync_copy(zbuf_v, acc_sh.at[pl.ds(row0, MB)])

    plsc.subcore_barrier()

    edge_pass(mid_hbm.at[c])

    plsc.subcore_barrier()

    @pl.loop(0, SLICE, step=MB)
    def _(r):
        pltpu.sync_copy(acc_sh.at[pl.ds(sl + r, MB)],
                        acc_hbm.at[c, pl.ds(sl + r, MB)])


def _pre_body(hists_ref, x_ref, oa_ref, ob_ref, deg_ref, rdeg_ref):
    deg = jnp.sum(hists_ref[...], axis=0) + 1.0
    dinv = lax.rsqrt(deg)
    u0 = dinv * x_ref[...]
    oa_ref[...] = u0[:, :F]
    ob_ref[...] = u0[:, F:]
    deg_ref[...] = deg
    rdeg_ref[...] = 1.0 / deg


def _cat(p):
    return jnp.concatenate([p[0], p[1]], axis=-1)


def _mlp_body(deg_ref, acc_ref, mid_ref, w1_ref, b1_ref, w2_ref,
              oa_ref, ob_ref):
    dinv = lax.rsqrt(deg_ref[...])
    hin = dinv * (_cat(acc_ref[...]) + _cat(mid_ref[...]))
    h = jnp.dot(hin, w1_ref[...], preferred_element_type=jnp.float32)
    h = jnp.maximum(h + b1_ref[...], 0.0)
    y = jnp.dot(h, w2_ref[...], preferred_element_type=jnp.float32)
    u2 = dinv * y
    oa_ref[...] = u2[:, :F]
    ob_ref[...] = u2[:, F:]


def _final_body(deg_ref, acc_ref, mid_ref, b2_ref, o_ref):
    dinv = lax.rsqrt(deg_ref[...])
    z = dinv * (_cat(acc_ref[...]) + _cat(mid_ref[...])) + b2_ref[...]
    m = jnp.max(z, axis=1, keepdims=True)
    e = jnp.exp(z - m)
    lse = jnp.log(jnp.sum(e, axis=1, keepdims=True))
    o_ref[...] = z - m - lse


_deg_spec = pl.BlockSpec((RB, 1), lambda i: (i, 0))
_row_spec = pl.BlockSpec((RB, D), lambda i: (i, 0))
_half_spec = pl.BlockSpec((RB, F), lambda i: (i, 0))
_p_spec = pl.BlockSpec((NC, RB, F), lambda i: (0, i, 0))

_tc_pre = pl.pallas_call(
    _pre_body, grid=(GRID,),
    in_specs=[pl.BlockSpec((NW, RB, 1), lambda i: (0, i, 0)), _row_spec],
    out_specs=[_half_spec, _half_spec, _deg_spec, _deg_spec],
    out_shape=[jax.ShapeDtypeStruct((NPAD, F), jnp.float32),
               jax.ShapeDtypeStruct((NPAD, F), jnp.float32),
               jax.ShapeDtypeStruct((NPAD, 1), jnp.float32),
               jax.ShapeDtypeStruct((NPAD, 1), jnp.float32)])

_tc_mlp = pl.pallas_call(
    _mlp_body, grid=(GRID,),
    in_specs=[_deg_spec, _p_spec, _p_spec,
              pl.BlockSpec((D, 256), lambda i: (0, 0)),
              pl.BlockSpec((1, 256), lambda i: (0, 0)),
              pl.BlockSpec((256, D), lambda i: (0, 0))],
    out_specs=[_half_spec, _half_spec],
    out_shape=[jax.ShapeDtypeStruct((NPAD, F), jnp.float32),
               jax.ShapeDtypeStruct((NPAD, F), jnp.float32)])

_tc_final = pl.pallas_call(
    _final_body, grid=(GRID,),
    in_specs=[_deg_spec, _p_spec, _p_spec,
              pl.BlockSpec((1, D), lambda i: (0, 0))],
    out_specs=_row_spec,
    out_shape=jax.ShapeDtypeStruct((NPAD, D), jnp.float32))


def kernel(x, edge_index, W1, b1, W2, b2):
    src = edge_index[0]
    dst = edge_index[1]
    src2 = src.reshape(-1, CH)
    dst2 = dst.reshape(-1, CH)
    x_pad = jnp.pad(x, ((0, NPAD - N), (0, 0)))

    hists = _deg_sc(dst).reshape(NW, NPAD, 1)
    u_a, u_b, deg, rdeg = _tc_pre(hists, x_pad)
    rdeg_flat = rdeg.reshape(NPAD)

    acc, mid = _prop2_sc(src2, dst2, u_a, u_b, rdeg_flat)
    u2a, u2b = _tc_mlp(deg, acc, mid, W1.T, b1.reshape(1, -1), W2.T)
    acc, mid = _prop2_sc(src2, dst2, u2a, u2b, rdeg_flat)
    out = _tc_final(deg, acc, mid, b2.reshape(1, -1))
    return out[:N]


# speedup vs baseline: 23.9971x; 1.3151x over previous
import dataclasses
import functools

import jax
import jax.numpy as jnp
from jax import lax
from jax.experimental import pallas as pl
from jax.experimental.pallas import tpu as pltpu
from jax.experimental.pallas import tpu_sc as plsc

N = 10000
E = 320000
NPAD = 10240
D = 128
F = D // 2
NC, NS = 2, 16
NW = NC * NS
CH = 125
NBUF = 4
EPT = E // NS
NCH = EPT // CH
DCH = 80
DEPT = E // NW
SLICE = NPAD // NS
ACC_R = 10112
SLICE_P = SLICE
MB = 64
RB = 1024
GRID = NPAD // RB

_mesh = plsc.VectorSubcoreMesh(core_axis_name="c", subcore_axis_name="s")

_sc_params = pltpu.CompilerParams()
if "needs_layout_passes" in pltpu.CompilerParams.__dataclass_fields__:
    _sc_params = dataclasses.replace(_sc_params, needs_layout_passes=False)
_sc_flat_params = dataclasses.replace(_sc_params, use_tc_tiling_on_sc=False)


@functools.partial(
    pl.kernel,
    out_type=jax.ShapeDtypeStruct((NC * NPAD,), jnp.float32),
    mesh=_mesh,
    compiler_params=_sc_flat_params,
    scratch_types=[
        pltpu.VMEM((NPAD,), jnp.float32),
        pltpu.VMEM((DCH,), jnp.int32),
        pltpu.VMEM((SLICE,), jnp.float32),
        pltpu.VMEM((SLICE,), jnp.float32),
        pltpu.VMEM_SHARED((NS, NPAD // 8), jnp.float32),
    ],
)
def _deg_sc(dst_hbm, deg_hbm, hist_v, dstb_v, acc_v, tmp_v, hists_sh):
    c = lax.axis_index("c")
    s = lax.axis_index("s")
    wid = c * NS + s
    zero16 = jnp.zeros((16,), jnp.float32)
    ones16 = jnp.ones((16,), jnp.float32)

    @pl.loop(0, NPAD, step=16)
    def _(i):
        hist_v[pl.ds(i, 16)] = zero16

    base = wid * DEPT

    @pl.loop(0, DEPT, step=DCH)
    def _(j):
        pltpu.sync_copy(dst_hbm.at[pl.ds(base + j, DCH)], dstb_v)

        @pl.loop(0, DCH, step=16)
        def _(k):
            plsc.addupdate_scatter(hist_v, [dstb_v[pl.ds(k, 16)]], ones16)

    hsl = NPAD // 8
    tsl = hsl // NS
    for half in range(8):
        pltpu.sync_copy(hist_v.at[pl.ds(half * hsl, hsl)], hists_sh.at[s])
        plsc.subcore_barrier()

        @pl.loop(0, tsl, step=16)
        def _(k):
            acc_v[pl.ds(k, 16)] = zero16

        @pl.loop(0, NS)
        def _(r):
            pltpu.sync_copy(hists_sh.at[r, pl.ds(s * tsl, tsl)],
                            tmp_v.at[pl.ds(0, tsl)])

            @pl.loop(0, tsl, step=16)
            def _(k):
                acc_v[pl.ds(k, 16)] = (acc_v[pl.ds(k, 16)]
                                       + tmp_v[pl.ds(k, 16)])

        pltpu.sync_copy(
            acc_v.at[pl.ds(0, tsl)],
            deg_hbm.at[pl.ds(c * NPAD + half * hsl + s * tsl, tsl)])
        plsc.subcore_barrier()


def _rsqrt16(d):
    i = plsc.bitcast(d, jnp.int32)
    i = jnp.int32(0x5F3759DF) - lax.shift_right_logical(i, 1)
    y = plsc.bitcast(i, jnp.float32)
    half_d = 0.5 * d
    for _ in range(4):
        y = y * (1.5 - half_d * y * y)
    return y


@functools.partial(
    pl.kernel,
    out_type=jax.ShapeDtypeStruct((NC, NPAD, F), jnp.float32),
    mesh=_mesh,
    compiler_params=_sc_flat_params,
    scratch_types=[
        pltpu.VMEM((NCH, CH), jnp.int32),
        pltpu.VMEM((NCH, CH), jnp.int32),
        pltpu.VMEM((CH, F), jnp.float32),
        pltpu.VMEM((CH, F), jnp.float32),
        pltpu.VMEM((CH, F), jnp.float32),
        pltpu.VMEM((CH, F), jnp.float32),
        pltpu.VMEM((MB, F), jnp.float32),
        pltpu.VMEM((MB, F), jnp.float32),
        pltpu.VMEM((MB, F), jnp.float32),
        pltpu.VMEM((MB, F), jnp.float32),
        pltpu.VMEM((SLICE,), jnp.float32),
        pltpu.VMEM((SLICE,), jnp.float32),
        pltpu.VMEM_SHARED((ACC_R, F), jnp.float32),
    ] + [pltpu.SemaphoreType.DMA] * (2 * NBUF),
)
def _prop2_sc(src_hbm, dst_hbm, xa_hbm, xb_hbm, degp_hbm,
              w_hbm,
              src_v, dst_v, r0, r1, r2, r3, zbuf_v, accb_v, rowb_v, resb_v,
              dinv_v, rdeg_v, acc_sh, g0, g1, g2, g3, s0, s1, s2, s3):
    c = lax.axis_index("c")
    s = lax.axis_index("s")
    rows = (r0, r1, r2, r3)
    gsem = (g0, g1, g2, g3)
    ssem = (s0, s1, s2, s3)
    zero16 = jnp.zeros((16,), jnp.float32)
    sl = s * SLICE_P

    pltpu.async_copy(src_hbm.at[pl.ds(s * NCH, NCH)], src_v, g0)
    pltpu.async_copy(dst_hbm.at[pl.ds(s * NCH, NCH)], dst_v, g1)

    @pl.loop(0, MB)
    def _(r):
        @pl.loop(0, F, step=16)
        def _(k):
            zbuf_v[r, pl.ds(k, 16)] = zero16

    @pl.loop(0, SLICE_P, step=MB)
    def _(r):
        @pl.when(sl + r + MB <= ACC_R)
        def _():
            pltpu.sync_copy(zbuf_v, acc_sh.at[pl.ds(sl + r, MB)])

    pltpu.sync_copy(degp_hbm.at[pl.ds(sl, SLICE)], dinv_v)
    pltpu.sync_copy(degp_hbm.at[pl.ds(NPAD + sl, SLICE)], rdeg_v)

    @pl.loop(0, SLICE, step=16)
    def _(k):
        deg = dinv_v[pl.ds(k, 16)] + rdeg_v[pl.ds(k, 16)] + 1.0
        y = _rsqrt16(deg)
        dinv_v[pl.ds(k, 16)] = y
        rdeg_v[pl.ds(k, 16)] = y * y

    @pl.loop(0, SLICE_P, step=MB)
    def _(r):
        row0 = sl + r

        @pl.when(c == 0)
        def _():
            pltpu.sync_copy(xa_hbm.at[pl.ds(row0, MB)], rowb_v)

        @pl.when(c == 1)
        def _():
            pltpu.sync_copy(xb_hbm.at[pl.ds(row0, MB)], rowb_v)

        @pl.loop(0, MB)
        def _(i):
            sc16 = plsc.load_gather(dinv_v, [jnp.full((16,), r + i,
                                                      jnp.int32)])
            for k in range(0, F, 16):
                resb_v[i, pl.ds(k, 16)] = rowb_v[i, pl.ds(k, 16)] * sc16

        pltpu.sync_copy(resb_v, w_hbm.at[c, pl.ds(row0, MB)])

    pltpu.make_async_copy(src_hbm.at[pl.ds(s * NCH, NCH)], src_v, g0).wait()
    pltpu.make_async_copy(dst_hbm.at[pl.ds(s * NCH, NCH)], dst_v, g1).wait()
    plsc.subcore_barrier()

    def edge_pass(table):
        for b in range(NBUF):
            pltpu.async_copy(table.at[src_v.at[b]], rows[b], gsem[b])

        @pl.loop(0, NCH, step=NBUF)
        def _(j):
            for b in range(NBUF):
                jj = j + b
                pltpu.make_async_copy(table.at[src_v.at[jj]], rows[b],
                                      gsem[b]).wait()
                pltpu.async_copy(rows[b], acc_sh.at[dst_v.at[jj]], ssem[b],
                                 add=True)

            for b in range(NBUF):
                jj = j + b
                pltpu.make_async_copy(rows[b], acc_sh.at[dst_v.at[jj]],
                                      ssem[b]).wait()

                @pl.when(jj + NBUF < NCH)
                def _():
                    pltpu.async_copy(table.at[src_v.at[jj + NBUF]], rows[b],
                                     gsem[b])

    def scale_phase(row_table, scale_v, out):
        @pl.loop(0, SLICE_P, step=MB)
        def _(r):
          row0 = sl + r

          @pl.when(row0 + MB <= ACC_R)
          def _():
            pltpu.sync_copy(acc_sh.at[pl.ds(row0, MB)], accb_v)
            pltpu.sync_copy(row_table.at[pl.ds(row0, MB)], rowb_v)

            @pl.loop(0, MB)
            def _(i):
                sc16 = plsc.load_gather(scale_v, [jnp.full((16,), r + i,
                                                           jnp.int32)])
                for k in range(0, F, 16):
                    resb_v[i, pl.ds(k, 16)] = (
                        accb_v[i, pl.ds(k, 16)]
                        + rowb_v[i, pl.ds(k, 16)]) * sc16

            pltpu.sync_copy(resb_v, out.at[pl.ds(row0, MB)])
            pltpu.sync_copy(zbuf_v, acc_sh.at[pl.ds(row0, MB)])

    edge_pass(w_hbm.at[c])
    plsc.subcore_barrier()
    scale_phase(w_hbm.at[c], rdeg_v, w_hbm.at[c])
    plsc.subcore_barrier()
    edge_pass(w_hbm.at[c])
    plsc.subcore_barrier()
    scale_phase(w_hbm.at[c], dinv_v, w_hbm.at[c])


def _cat(p):
    return jnp.concatenate([p[0], p[1]], axis=-1)


def _mlp_body(z_ref, w1_ref, b1_ref, w2_ref, oa_ref, ob_ref):
    hin = _cat(z_ref[...])
    h = jnp.dot(hin, w1_ref[...], preferred_element_type=jnp.float32)
    h = jnp.maximum(h + b1_ref[...], 0.0)
    y = jnp.dot(h, w2_ref[...], preferred_element_type=jnp.float32)
    oa_ref[...] = y[:, :F]
    ob_ref[...] = y[:, F:]


def _final_body(z_ref, b2_ref, o_ref):
    z = _cat(z_ref[...]) + b2_ref[...]
    m = jnp.max(z, axis=1, keepdims=True)
    e = jnp.exp(z - m)
    lse = jnp.log(jnp.sum(e, axis=1, keepdims=True))
    o_ref[...] = z - m - lse


_row_spec = pl.BlockSpec((RB, D), lambda i: (i, 0))
_half_spec = pl.BlockSpec((RB, F), lambda i: (i, 0))
_p_spec = pl.BlockSpec((NC, RB, F), lambda i: (0, i, 0))

_tc_mlp = pl.pallas_call(
    _mlp_body, grid=(GRID,),
    in_specs=[_p_spec,
              pl.BlockSpec((D, 256), lambda i: (0, 0)),
              pl.BlockSpec((1, 256), lambda i: (0, 0)),
              pl.BlockSpec((256, D), lambda i: (0, 0))],
    out_specs=[_half_spec, _half_spec],
    out_shape=[jax.ShapeDtypeStruct((NPAD, F), jnp.float32),
               jax.ShapeDtypeStruct((NPAD, F), jnp.float32)])

_tc_final = pl.pallas_call(
    _final_body, grid=(GRID,),
    in_specs=[_p_spec, pl.BlockSpec((1, D), lambda i: (0, 0))],
    out_specs=_row_spec,
    out_shape=jax.ShapeDtypeStruct((NPAD, D), jnp.float32))


def kernel(x, edge_index, W1, b1, W2, b2):
    src = edge_index[0]
    dst = edge_index[1]
    src2 = src.reshape(-1, CH)
    dst2 = dst.reshape(-1, CH)
    x_pad = jnp.pad(x, ((0, NPAD - N), (0, 0)))
    xa = x_pad[:, :F]
    xb = x_pad[:, F:]

    degp = _deg_sc(dst)
    w1t = W1.T
    b1r = b1.reshape(1, -1)
    w2t = W2.T

    z1 = _prop2_sc(src2, dst2, xa, xb, degp)
    ya, yb = _tc_mlp(z1, w1t, b1r, w2t)
    z2 = _prop2_sc(src2, dst2, ya, yb, degp)
    out = _tc_final(z2, b2.reshape(1, -1))
    return out[:N]


# speedup vs baseline: 25.0558x; 1.0441x over previous
import dataclasses
import functools

import jax
import jax.numpy as jnp
from jax import lax
from jax.experimental import pallas as pl
from jax.experimental.pallas import tpu as pltpu
from jax.experimental.pallas import tpu_sc as plsc

N = 10000
E = 320000
NPAD = 10240
D = 128
F = D // 2
NC, NS = 2, 16
NW = NC * NS
CH = 125
NBUF = 4
EPT = E // NS
NCH = EPT // CH
DCH = 80
DEPT = E // NW
SLICE = NPAD // NS
ACC_R = 10112
SLICE_P = SLICE
MB = 64
RB = 1024
GRID = NPAD // RB

_mesh = plsc.VectorSubcoreMesh(core_axis_name="c", subcore_axis_name="s")

_sc_params = pltpu.CompilerParams()
if "needs_layout_passes" in pltpu.CompilerParams.__dataclass_fields__:
    _sc_params = dataclasses.replace(_sc_params, needs_layout_passes=False)
_sc_flat_params = dataclasses.replace(_sc_params, use_tc_tiling_on_sc=False)


@functools.partial(
    pl.kernel,
    out_type=jax.ShapeDtypeStruct((NC * NPAD,), jnp.float32),
    mesh=_mesh,
    compiler_params=_sc_flat_params,
    scratch_types=[
        pltpu.VMEM((NPAD,), jnp.float32),
        pltpu.VMEM((DCH,), jnp.int32),
        pltpu.VMEM((DCH,), jnp.int32),
        pltpu.VMEM((SLICE,), jnp.float32),
        pltpu.VMEM((SLICE,), jnp.float32),
        pltpu.VMEM_SHARED((NS, NPAD), jnp.float32),
        pltpu.SemaphoreType.DMA,
        pltpu.SemaphoreType.DMA,
    ],
)
def _deg_sc(dst_hbm, deg_hbm, hist_v, dstb_v, dstb2_v, acc_v, tmp_v,
            hists_sh, d0, d1):
    c = lax.axis_index("c")
    s = lax.axis_index("s")
    wid = c * NS + s
    zero16 = jnp.zeros((16,), jnp.float32)
    ones16 = jnp.ones((16,), jnp.float32)

    @pl.loop(0, NPAD, step=16)
    def _(i):
        hist_v[pl.ds(i, 16)] = zero16

    base = wid * DEPT
    bufs = (dstb_v, dstb2_v)
    sems = (d0, d1)

    def count(buf):
        @pl.loop(0, DCH, step=16)
        def _(k):
            plsc.addupdate_scatter(hist_v, [buf[pl.ds(k, 16)]], ones16)

    pltpu.async_copy(dst_hbm.at[pl.ds(base, DCH)], dstb_v, d0)

    @pl.loop(0, DEPT - DCH, step=2 * DCH)
    def _(j):
        for h in range(2):
            off = base + j + h * DCH
            nxt = off + DCH
            pltpu.make_async_copy(dst_hbm.at[pl.ds(off, DCH)], bufs[h],
                                  sems[h]).wait()
            pltpu.async_copy(dst_hbm.at[pl.ds(nxt, DCH)], bufs[1 - h],
                             sems[1 - h])
            count(bufs[h])

    pltpu.make_async_copy(dst_hbm.at[pl.ds(base + DEPT - DCH, DCH)], dstb_v,
                          d0).wait()
    count(dstb_v)

    hsl = NPAD
    tsl = hsl // NS
    for half in range(1):
        pltpu.sync_copy(hist_v.at[pl.ds(half * hsl, hsl)], hists_sh.at[s])
        plsc.subcore_barrier()

        @pl.loop(0, tsl, step=16)
        def _(k):
            acc_v[pl.ds(k, 16)] = zero16

        @pl.loop(0, NS)
        def _(r):
            pltpu.sync_copy(hists_sh.at[r, pl.ds(s * tsl, tsl)],
                            tmp_v.at[pl.ds(0, tsl)])

            @pl.loop(0, tsl, step=16)
            def _(k):
                acc_v[pl.ds(k, 16)] = (acc_v[pl.ds(k, 16)]
                                       + tmp_v[pl.ds(k, 16)])

        pltpu.sync_copy(
            acc_v.at[pl.ds(0, tsl)],
            deg_hbm.at[pl.ds(c * NPAD + half * hsl + s * tsl, tsl)])
        plsc.subcore_barrier()


def _rsqrt16(d):
    i = plsc.bitcast(d, jnp.int32)
    i = jnp.int32(0x5F3759DF) - lax.shift_right_logical(i, 1)
    y = plsc.bitcast(i, jnp.float32)
    half_d = 0.5 * d
    for _ in range(4):
        y = y * (1.5 - half_d * y * y)
    return y


@functools.partial(
    pl.kernel,
    out_type=jax.ShapeDtypeStruct((NC, NPAD, F), jnp.float32),
    mesh=_mesh,
    compiler_params=_sc_flat_params,
    scratch_types=[
        pltpu.VMEM((NCH, CH), jnp.int32),
        pltpu.VMEM((NCH, CH), jnp.int32),
        pltpu.VMEM((CH, F), jnp.float32),
        pltpu.VMEM((CH, F), jnp.float32),
        pltpu.VMEM((CH, F), jnp.float32),
        pltpu.VMEM((CH, F), jnp.float32),
        pltpu.VMEM((MB, F), jnp.float32),
        pltpu.VMEM((MB, F), jnp.float32),
        pltpu.VMEM((MB, F), jnp.float32),
        pltpu.VMEM((MB, F), jnp.float32),
        pltpu.VMEM((SLICE,), jnp.float32),
        pltpu.VMEM((SLICE,), jnp.float32),
        pltpu.VMEM_SHARED((ACC_R, F), jnp.float32),
    ] + [pltpu.SemaphoreType.DMA] * (2 * NBUF),
)
def _prop2_sc(src_hbm, dst_hbm, xa_hbm, xb_hbm, degp_hbm,
              w_hbm,
              src_v, dst_v, r0, r1, r2, r3, zbuf_v, accb_v, rowb_v, resb_v,
              dinv_v, rdeg_v, acc_sh, g0, g1, g2, g3, s0, s1, s2, s3):
    c = lax.axis_index("c")
    s = lax.axis_index("s")
    rows = (r0, r1, r2, r3)
    gsem = (g0, g1, g2, g3)
    ssem = (s0, s1, s2, s3)
    zero16 = jnp.zeros((16,), jnp.float32)
    sl = s * SLICE_P

    pltpu.async_copy(src_hbm.at[pl.ds(s * NCH, NCH)], src_v, g0)
    pltpu.async_copy(dst_hbm.at[pl.ds(s * NCH, NCH)], dst_v, g1)

    @pl.loop(0, MB)
    def _(r):
        @pl.loop(0, F, step=16)
        def _(k):
            zbuf_v[r, pl.ds(k, 16)] = zero16

    @pl.loop(0, SLICE_P, step=MB)
    def _(r):
        @pl.when(sl + r + MB <= ACC_R)
        def _():
            pltpu.async_copy(zbuf_v, acc_sh.at[pl.ds(sl + r, MB)], s0)

    @pl.loop(0, SLICE_P, step=MB)
    def _(r):
        @pl.when(sl + r + MB <= ACC_R)
        def _():
            pltpu.make_async_copy(zbuf_v, acc_sh.at[pl.ds(sl + r, MB)],
                                  s0).wait()

    pltpu.sync_copy(degp_hbm.at[pl.ds(sl, SLICE)], dinv_v)
    pltpu.sync_copy(degp_hbm.at[pl.ds(NPAD + sl, SLICE)], rdeg_v)

    @pl.loop(0, SLICE, step=16)
    def _(k):
        deg = dinv_v[pl.ds(k, 16)] + rdeg_v[pl.ds(k, 16)] + 1.0
        y = _rsqrt16(deg)
        dinv_v[pl.ds(k, 16)] = y
        rdeg_v[pl.ds(k, 16)] = y * y

    @pl.loop(0, SLICE_P, step=MB)
    def _(r):
        row0 = sl + r

        @pl.when(c == 0)
        def _():
            pltpu.sync_copy(xa_hbm.at[pl.ds(row0, MB)], rowb_v)

        @pl.when(c == 1)
        def _():
            pltpu.sync_copy(xb_hbm.at[pl.ds(row0, MB)], rowb_v)

        @pl.loop(0, MB)
        def _(i):
            sc16 = plsc.load_gather(dinv_v, [jnp.full((16,), r + i,
                                                      jnp.int32)])
            for k in range(0, F, 16):
                resb_v[i, pl.ds(k, 16)] = rowb_v[i, pl.ds(k, 16)] * sc16

        pltpu.sync_copy(resb_v, w_hbm.at[c, pl.ds(row0, MB)])

    pltpu.make_async_copy(src_hbm.at[pl.ds(s * NCH, NCH)], src_v, g0).wait()
    pltpu.make_async_copy(dst_hbm.at[pl.ds(s * NCH, NCH)], dst_v, g1).wait()
    plsc.subcore_barrier()

    def edge_pass(table):
        for b in range(NBUF):
            pltpu.async_copy(table.at[src_v.at[b]], rows[b], gsem[b])

        @pl.loop(0, NCH, step=NBUF)
        def _(j):
            for b in range(NBUF):
                jj = j + b
                pltpu.make_async_copy(table.at[src_v.at[jj]], rows[b],
                                      gsem[b]).wait()
                pltpu.async_copy(rows[b], acc_sh.at[dst_v.at[jj]], ssem[b],
                                 add=True)

            for b in range(NBUF):
                jj = j + b
                pltpu.make_async_copy(rows[b], acc_sh.at[dst_v.at[jj]],
                                      ssem[b]).wait()

                @pl.when(jj + NBUF < NCH)
                def _():
                    pltpu.async_copy(table.at[src_v.at[jj + NBUF]], rows[b],
                                     gsem[b])

    def scale_phase(row_table, scale_v, out):
        @pl.loop(0, SLICE_P, step=MB)
        def _(r):
          row0 = sl + r

          @pl.when(row0 + MB <= ACC_R)
          def _():
            pltpu.async_copy(acc_sh.at[pl.ds(row0, MB)], accb_v, g0)
            pltpu.async_copy(row_table.at[pl.ds(row0, MB)], rowb_v, g1)
            pltpu.make_async_copy(acc_sh.at[pl.ds(row0, MB)], accb_v,
                                  g0).wait()
            pltpu.make_async_copy(row_table.at[pl.ds(row0, MB)], rowb_v,
                                  g1).wait()

            @pl.loop(0, MB)
            def _(i):
                sc16 = plsc.load_gather(scale_v, [jnp.full((16,), r + i,
                                                           jnp.int32)])
                for k in range(0, F, 16):
                    resb_v[i, pl.ds(k, 16)] = (
                        accb_v[i, pl.ds(k, 16)]
                        + rowb_v[i, pl.ds(k, 16)]) * sc16

            pltpu.async_copy(resb_v, out.at[pl.ds(row0, MB)], s1)
            pltpu.async_copy(zbuf_v, acc_sh.at[pl.ds(row0, MB)], s2)
            pltpu.make_async_copy(resb_v, out.at[pl.ds(row0, MB)], s1).wait()
            pltpu.make_async_copy(zbuf_v, acc_sh.at[pl.ds(row0, MB)],
                                  s2).wait()

    edge_pass(w_hbm.at[c])
    plsc.subcore_barrier()
    scale_phase(w_hbm.at[c], rdeg_v, w_hbm.at[c])
    plsc.subcore_barrier()
    edge_pass(w_hbm.at[c])
    plsc.subcore_barrier()
    scale_phase(w_hbm.at[c], dinv_v, w_hbm.at[c])


def _cat(p):
    return jnp.concatenate([p[0], p[1]], axis=-1)


def _mlp_body(z_ref, w1_ref, b1_ref, w2_ref, oa_ref, ob_ref):
    hin = _cat(z_ref[...])
    h = jnp.dot(hin, w1_ref[...], preferred_element_type=jnp.float32)
    h = jnp.maximum(h + b1_ref[...], 0.0)
    y = jnp.dot(h, w2_ref[...], preferred_element_type=jnp.float32)
    oa_ref[...] = y[:, :F]
    ob_ref[...] = y[:, F:]


def _final_body(z_ref, b2_ref, o_ref):
    z = _cat(z_ref[...]) + b2_ref[...]
    m = jnp.max(z, axis=1, keepdims=True)
    e = jnp.exp(z - m)
    lse = jnp.log(jnp.sum(e, axis=1, keepdims=True))
    o_ref[...] = z - m - lse


_row_spec = pl.BlockSpec((RB, D), lambda i: (i, 0))
_half_spec = pl.BlockSpec((RB, F), lambda i: (i, 0))
_p_spec = pl.BlockSpec((NC, RB, F), lambda i: (0, i, 0))

_tc_mlp = pl.pallas_call(
    _mlp_body, grid=(GRID,),
    in_specs=[_p_spec,
              pl.BlockSpec((D, 256), lambda i: (0, 0)),
              pl.BlockSpec((1, 256), lambda i: (0, 0)),
              pl.BlockSpec((256, D), lambda i: (0, 0))],
    out_specs=[_half_spec, _half_spec],
    out_shape=[jax.ShapeDtypeStruct((NPAD, F), jnp.float32),
               jax.ShapeDtypeStruct((NPAD, F), jnp.float32)])

_tc_final = pl.pallas_call(
    _final_body, grid=(GRID,),
    in_specs=[_p_spec, pl.BlockSpec((1, D), lambda i: (0, 0))],
    out_specs=_row_spec,
    out_shape=jax.ShapeDtypeStruct((NPAD, D), jnp.float32))


def kernel(x, edge_index, W1, b1, W2, b2):
    src = edge_index[0]
    dst = edge_index[1]
    src2 = src.reshape(-1, CH)
    dst2 = dst.reshape(-1, CH)
    x_pad = jnp.pad(x, ((0, NPAD - N), (0, 0)))
    xa = x_pad[:, :F]
    xb = x_pad[:, F:]

    degp = _deg_sc(dst)
    w1t = W1.T
    b1r = b1.reshape(1, -1)
    w2t = W2.T

    z1 = _prop2_sc(src2, dst2, xa, xb, degp)
    ya, yb = _tc_mlp(z1, w1t, b1r, w2t)
    z2 = _prop2_sc(src2, dst2, ya, yb, degp)
    out = _tc_final(z2, b2.reshape(1, -1))
    return out[:N]
